# baseline (speedup 1.0000x reference)
import jax
import jax.numpy as jnp
from jax import lax
from jax.experimental import pallas as pl
from jax.experimental.pallas import tpu as pltpu
from jax.experimental.pallas import tpu_sc as plsc

N = 10000
D = 128
E = 320000
NC = 2
NS = 16
L = 16
WIN = 96
NPAD = N + 112
NW = NC * NS
NWIN = 108
EPW = NWIN * WIN
EP = EPW * NW
RSL = NPAD // NS
NEG = -1.0e9


def _leaky(z):
    return jnp.maximum(z, 0.2 * z)


_ZCH = (RSL + WIN - 1) // WIN
_ZREM = RSL - (_ZCH - 1) * WIN


def _sc_body(h_hbm, ssrc_hbm, sdst_hbm, smax_hbm, srcp_hbm, dstp_hbm,
             z2_hbm, z1_hbm, num_out, den_out,
             ssrc_t, sdst_t, smax_t, idx_s, idx_d, ee_v, rows_v, zbuf,
             num_sh, den_sh, sem_i, sem_g, sem_s):
    cid = lax.axis_index("c")
    sid = lax.axis_index("s")
    wid = sid * NC + cid

    pltpu.sync_copy(ssrc_hbm, ssrc_t)
    pltpu.sync_copy(sdst_hbm, sdst_t)
    pltpu.sync_copy(smax_hbm.at[pl.ds(0, L)], smax_t)

    pltpu.sync_copy(z1_hbm.at[pl.ds(sid * RSL, RSL)], zbuf)
    pltpu.sync_copy(zbuf, den_sh.at[pl.ds(sid * RSL, RSL)])
    pltpu.sync_copy(z2_hbm.at[pl.ds(0, WIN)], rows_v.at[0])
    for k in range(_ZCH):
        w = WIN if k < _ZCH - 1 else _ZREM
        pltpu.sync_copy(rows_v.at[0, pl.ds(0, w)],
                        num_sh.at[pl.ds(sid * RSL + k * WIN, w)])
    plsc.subcore_barrier()

    base = wid * EPW
    smax_v = smax_t[...]

    def fetch_idx(wi, slot):
        off = base + wi * WIN
        pltpu.async_copy(srcp_hbm.at[pl.ds(off, WIN)], idx_s.at[slot],
                         sem_i.at[slot])
        pltpu.async_copy(dstp_hbm.at[pl.ds(off, WIN)], idx_d.at[slot],
                         sem_i.at[slot])

    def wait_idx(slot):
        pltpu.make_async_copy(srcp_hbm.at[pl.ds(0, WIN)], idx_s.at[slot],
                              sem_i.at[slot]).wait()
        pltpu.make_async_copy(dstp_hbm.at[pl.ds(0, WIN)], idx_d.at[slot],
                              sem_i.at[slot]).wait()

    def start_gather(slot, b):
        pltpu.async_copy(h_hbm.at[idx_s.at[slot]], rows_v.at[b], sem_g)

    def wait_gather():
        pltpu.make_async_copy(z2_hbm.at[pl.ds(0, WIN)], rows_v.at[0],
                              sem_g).wait()

    def start_scatter(slot, b):
        pltpu.async_copy(rows_v.at[b], num_sh.at[idx_d.at[slot]], sem_s,
                         add=True)
        pltpu.async_copy(ee_v.at[b], den_sh.at[idx_d.at[slot]], sem_s,
                         add=True)

    def wait_scatter():
        pltpu.make_async_copy(z2_hbm.at[pl.ds(0, WIN)], rows_v.at[0],
                              sem_s).wait()
        pltpu.make_async_copy(z1_hbm.at[pl.ds(0, WIN)], ee_v.at[0],
                              sem_s).wait()

    def compute_ee(slot, b):
        def grp(g, c2):
            sv = idx_s[slot, pl.ds(g * L, L)]
            dv = idx_d[slot, pl.ds(g * L, L)]
            a = plsc.load_gather(ssrc_t, [sv])
            bb = plsc.load_gather(sdst_t, [dv])
            cc = _leaky(smax_v + bb)
            ee_v[b, pl.ds(g * L, L)] = jnp.exp(_leaky(a + bb) - cc)
            return c2

        lax.fori_loop(0, WIN // L, grp, 0)

    def scale_rows(b):
        def grp(g, c2):
            ev = ee_v[b, pl.ds(g * L, L)]
            for j in range(L):
                w = jnp.full((L,), ev[j], jnp.float32)
                i = g * L + j
                for f in range(D // L):
                    rows_v[b, i, pl.ds(f * L, L)] = (
                        rows_v[b, i, pl.ds(f * L, L)] * w)
            return c2

        lax.fori_loop(0, WIN // L, grp, 0)

    fetch_idx(0, 0)
    fetch_idx(1, 1)
    wait_idx(0)
    start_gather(0, 0)

    def step(wp, carry):
        last_wp = wp >= NWIN // 4 - 1
        for b4 in range(4):
            b = b4 % 2
            nb = 1 - b
            nslot = (b4 + 1) % 4
            fslot = (b4 + 2) % 4
            compute_ee(b4, b)
            wait_gather()
            scale_rows(b)

            def drain():
                wait_scatter()

            if b4 == 0:
                pl.when(wp >= 1)(drain)
            else:
                drain()

            start_scatter(b4, b)

            def advance():
                wait_idx(nslot)
                start_gather(nslot, nb)

            if b4 == 3:
                pl.when(jnp.logical_not(last_wp))(advance)
            else:
                advance()

            def prefetch():
                fetch_idx(wp * 4 + b4 + 2, fslot)

            if b4 >= 2:
                pl.when(jnp.logical_not(last_wp))(prefetch)
            else:
                prefetch()
        return carry

    lax.fori_loop(0, NWIN // 4, step, 0)
    wait_scatter()
    plsc.subcore_barrier()

    for k in range(_ZCH):
        w = WIN if k < _ZCH - 1 else _ZREM
        pltpu.sync_copy(num_sh.at[pl.ds(sid * RSL + k * WIN, w)],
                        rows_v.at[0, pl.ds(0, w)])
        pltpu.sync_copy(rows_v.at[0, pl.ds(0, w)],
                        num_out.at[cid, pl.ds(sid * RSL + k * WIN, w)])
    pltpu.sync_copy(den_sh.at[pl.ds(sid * RSL, RSL)], zbuf)
    pltpu.sync_copy(zbuf, den_out.at[pl.ds(cid * NPAD + sid * RSL, RSL)])


_sc_layer = pl.kernel(
    _sc_body,
    out_type=[
        jax.ShapeDtypeStruct((NC, NPAD, D), jnp.float32),
        jax.ShapeDtypeStruct((NC * NPAD,), jnp.float32),
    ],
    mesh=plsc.VectorSubcoreMesh(core_axis_name="c", subcore_axis_name="s",
                                num_cores=NC, num_subcores=NS),
    compiler_params=pltpu.CompilerParams(needs_layout_passes=False),
    scratch_types=[
        pltpu.VMEM((NPAD,), jnp.float32),
        pltpu.VMEM((NPAD,), jnp.float32),
        pltpu.VMEM((L,), jnp.float32),
        pltpu.VMEM((4, WIN), jnp.int32),
        pltpu.VMEM((4, WIN), jnp.int32),
        pltpu.VMEM((2, WIN), jnp.float32),
        pltpu.VMEM((2, WIN, D), jnp.float32),
        pltpu.VMEM((RSL,), jnp.float32),
        pltpu.VMEM_SHARED((NPAD, D), jnp.float32),
        pltpu.VMEM_SHARED((NPAD,), jnp.float32),
        pltpu.SemaphoreType.DMA((4,)),
        pltpu.SemaphoreType.DMA,
        pltpu.SemaphoreType.DMA,
    ],
)


def _row_mask():
    rows = lax.broadcasted_iota(jnp.int32, (NPAD, 1), 0)
    return rows < N


def _attn_tables(h, asrc, adst):
    mask = _row_mask()
    ssrc = jnp.sum(h * asrc[None, :], axis=-1, keepdims=True)
    sdst = jnp.sum(h * adst[None, :], axis=-1, keepdims=True)
    ssrc = jnp.where(mask, ssrc, NEG)
    sdst = jnp.where(mask, sdst, 0.0)
    smax = jnp.max(ssrc)
    return ssrc, sdst, jnp.full((1, D), smax, jnp.float32)


def _tc_pre1_body(x_ref, w_ref, asrc_ref, adst_ref,
                  h_ref, ssrc_ref, sdst_ref, smax_ref):
    x = x_ref[...]
    h = jnp.dot(x, w_ref[...], preferred_element_type=jnp.float32)
    h_ref[...] = h
    ssrc, sdst, smax = _attn_tables(h, asrc_ref[...], adst_ref[...])
    ssrc_ref[...] = ssrc
    sdst_ref[...] = sdst
    smax_ref[...] = smax


def _gat_combine(num_ref, den_ref, b):
    num = num_ref[0] + num_ref[1]
    den = den_ref[0] + den_ref[1]
    gat = num / (den + 1e-16) + b[None, :]
    return jnp.maximum(gat, 0.0)


def _bn(x, g, b, m, v):
    return (x - m[None, :]) / jnp.sqrt(v[None, :] + 1e-5) * g[None, :] + b[None, :]


def _pre_next(xi, w_ref, asrc_ref, adst_ref, newprev_ref, h_ref, ssrc_ref,
              sdst_ref, smax_ref):
    xi = jnp.where(_row_mask(), xi, 0.0)
    newprev_ref[...] = xi
    h = jnp.dot(xi, w_ref[...], preferred_element_type=jnp.float32)
    h_ref[...] = h
    ssrc, sdst, smax = _attn_tables(h, asrc_ref[...], adst_ref[...])
    ssrc_ref[...] = ssrc
    sdst_ref[...] = sdst
    smax_ref[...] = smax


def _tc_mid1_body(num_ref, den_ref, cb_ref, bng_ref, bnb_ref, bnm_ref,
                  bnv_ref, w_ref, asrc_ref, adst_ref,
                  newprev_ref, h_ref, ssrc_ref, sdst_ref, smax_ref):
    act = _gat_combine(num_ref, den_ref, cb_ref[...])
    xi = _bn(act, bng_ref[...], bnb_ref[...], bnm_ref[...], bnv_ref[...])
    _pre_next(xi, w_ref, asrc_ref, adst_ref, newprev_ref, h_ref, ssrc_ref,
              sdst_ref, smax_ref)


def _tc_mid_body(num_ref, den_ref, prev_ref, cb_ref, bng_ref, bnb_ref,
                 bnm_ref, bnv_ref, pw_ref, pb_ref, w_ref, asrc_ref, adst_ref,
                 newprev_ref, h_ref, ssrc_ref, sdst_ref, smax_ref):
    act = _gat_combine(num_ref, den_ref, cb_ref[...])
    xi = _bn(act, bng_ref[...], bnb_ref[...], bnm_ref[...], bnv_ref[...])
    xi = xi + jnp.dot(prev_ref[...], pw_ref[...],
                      preferred_element_type=jnp.float32) + pb_ref[...][None, :]
    _pre_next(xi, w_ref, asrc_ref, adst_ref, newprev_ref, h_ref, ssrc_ref,
              sdst_ref, smax_ref)


def _tc_final_body(num_ref, den_ref, prev_ref, cb_ref, bng_ref, bnb_ref,
                   bnm_ref, bnv_ref, pw_ref, pb_ref, hw1_ref, hb1_ref,
                   hg_ref, hbb_ref, hm_ref, hv_ref, hw2_ref, hb2_ref,
                   out_ref):
    act = _gat_combine(num_ref, den_ref, cb_ref[...])
    xi = _bn(act, bng_ref[...], bnb_ref[...], bnm_ref[...], bnv_ref[...])
    xi = xi + jnp.dot(prev_ref[...], pw_ref[...],
                      preferred_element_type=jnp.float32) + pb_ref[...][None, :]
    xi = jnp.where(_row_mask(), xi, 0.0)
    g = jnp.sum(xi, axis=0, keepdims=True) / float(N)
    hh = jnp.dot(g, hw1_ref[...], preferred_element_type=jnp.float32)
    hh = jnp.maximum(hh + hb1_ref[...][None, :], 0.0)
    hh = _bn(hh, hg_ref[...], hbb_ref[...], hm_ref[...], hv_ref[...])
    out = jnp.dot(hh, hw2_ref[...], preferred_element_type=jnp.float32)
    out_ref[...] = out + hb2_ref[...][None, :]


_node_f32 = jax.ShapeDtypeStruct((NPAD, 1), jnp.float32)
_feat_f32 = jax.ShapeDtypeStruct((NPAD, D), jnp.float32)
_smax_f32 = jax.ShapeDtypeStruct((1, D), jnp.float32)

_tc_pre1 = pl.pallas_call(
    _tc_pre1_body,
    out_shape=[_feat_f32, _node_f32, _node_f32, _smax_f32],
)

_tc_mid1 = pl.pallas_call(
    _tc_mid1_body,
    out_shape=[_feat_f32, _feat_f32, _node_f32, _node_f32, _smax_f32],
)

_tc_mid = pl.pallas_call(
    _tc_mid_body,
    out_shape=[_feat_f32, _feat_f32, _node_f32, _node_f32, _smax_f32],
)

_tc_final = pl.pallas_call(
    _tc_final_body,
    out_shape=jax.ShapeDtypeStruct((1, 1), jnp.float32),
)


def kernel(x, edge_index, params):
    p = params
    xp = jnp.pad(x, ((0, NPAD - N), (0, 0)))
    src = edge_index[0]
    dst = edge_index[1]
    npad_e = EP - E
    pad_idx = N + (jnp.arange(npad_e, dtype=jnp.int32) % (NPAD - N))
    srcp = jnp.concatenate([src.astype(jnp.int32), pad_idx])
    dstp = jnp.concatenate([dst.astype(jnp.int32), pad_idx])
    z2 = jnp.zeros((NPAD, D), jnp.float32)
    z1 = jnp.zeros((NPAD,), jnp.float32)

    def flat(a):
        return a.reshape(NPAD)

    def d2(a):
        return a.reshape(NC, NPAD, 1)

    h, ssrc, sdst, smax = _tc_pre1(xp, p['conv1_W'], p['conv1_asrc'],
                                   p['conv1_adst'])
    num, den = _sc_layer(h, flat(ssrc), flat(sdst), smax.reshape(D), srcp,
                         dstp, z2, z1)

    prev, h, ssrc, sdst, smax = _tc_mid1(
        num, d2(den), p['conv1_b'],
        p['bn1_g'], p['bn1_b'], p['bn1_m'], p['bn1_v'],
        p['conv2_W'], p['conv2_asrc'], p['conv2_adst'])
    num, den = _sc_layer(h, flat(ssrc), flat(sdst), smax.reshape(D), srcp,
                         dstp, z2, z1)

    for i in range(3, 6):
        j = i - 1
        prev, h, ssrc, sdst, smax = _tc_mid(
            num, d2(den), prev, p['conv%d_b' % j],
            p['bn%d_g' % j], p['bn%d_b' % j], p['bn%d_m' % j], p['bn%d_v' % j],
            p['proj%d_W' % j], p['proj%d_b' % j],
            p['conv%d_W' % i], p['conv%d_asrc' % i], p['conv%d_adst' % i])
        num, den = _sc_layer(h, flat(ssrc), flat(sdst), smax.reshape(D), srcp,
                             dstp, z2, z1)

    out = _tc_final(
        num, d2(den), prev, p['conv5_b'],
        p['bn5_g'], p['bn5_b'], p['bn5_m'], p['bn5_v'],
        p['proj5_W'], p['proj5_b'],
        p['head_W1'], p['head_b1'],
        p['headbn_g'], p['headbn_b'], p['headbn_m'], p['headbn_v'],
        p['head_W2'], p['head_b2'])
    return out.reshape(-1)


# speedup vs baseline: 1.2532x; 1.2532x over previous
import jax
import jax.numpy as jnp
from jax import lax
from jax.experimental import pallas as pl
from jax.experimental.pallas import tpu as pltpu
from jax.experimental.pallas import tpu_sc as plsc

N = 10000
D = 128
E = 320000
NC = 2
NS = 16
L = 16
WIN = 48
NPAD = N + 112
NW = NC * NS
NWIN = 212
EPW = NWIN * WIN
EP = EPW * NW
RSL = NPAD // NS
NEG = -1.0e9


def _leaky(z):
    return jnp.maximum(z, 0.2 * z)


_ZCH = (RSL + WIN - 1) // WIN
_ZREM = RSL - (_ZCH - 1) * WIN


def _sc_body(h_hbm, ssrc_hbm, sdst_hbm, smax_hbm, srcp_hbm, dstp_hbm,
             z2_hbm, z1_hbm, num_out, den_out,
             ssrc_t, sdst_t, smax_t, idx_s, idx_d, ee_v, rows_v, zbuf,
             num_sh, den_sh, sem_i, sem_g, sem_s):
    cid = lax.axis_index("c")
    sid = lax.axis_index("s")
    wid = sid * NC + cid

    pltpu.sync_copy(ssrc_hbm, ssrc_t)
    pltpu.sync_copy(sdst_hbm, sdst_t)
    pltpu.sync_copy(smax_hbm.at[pl.ds(0, L)], smax_t)

    pltpu.sync_copy(z1_hbm.at[pl.ds(sid * RSL, RSL)], zbuf)
    pltpu.sync_copy(zbuf, den_sh.at[pl.ds(sid * RSL, RSL)])
    pltpu.sync_copy(z2_hbm.at[pl.ds(0, WIN)], rows_v.at[0])
    for k in range(_ZCH):
        w = WIN if k < _ZCH - 1 else _ZREM
        pltpu.sync_copy(rows_v.at[0, pl.ds(0, w)],
                        num_sh.at[pl.ds(sid * RSL + k * WIN, w)])
    plsc.subcore_barrier()

    base = wid * EPW
    smax_v = smax_t[...]

    def fetch_idx(wi, slot):
        off = base + wi * WIN
        pltpu.async_copy(srcp_hbm.at[pl.ds(off, WIN)], idx_s.at[slot],
                         sem_i.at[slot])
        pltpu.async_copy(dstp_hbm.at[pl.ds(off, WIN)], idx_d.at[slot],
                         sem_i.at[slot])

    def wait_idx(slot):
        pltpu.make_async_copy(srcp_hbm.at[pl.ds(0, WIN)], idx_s.at[slot],
                              sem_i.at[slot]).wait()
        pltpu.make_async_copy(dstp_hbm.at[pl.ds(0, WIN)], idx_d.at[slot],
                              sem_i.at[slot]).wait()

    def start_gather(slot, par):
        pltpu.async_copy(h_hbm.at[idx_s.at[slot]], rows_v.at[slot],
                         sem_g.at[par])

    def wait_gather(par):
        pltpu.make_async_copy(z2_hbm.at[pl.ds(0, WIN)], rows_v.at[0],
                              sem_g.at[par]).wait()

    def start_scatter(slot, par):
        pltpu.async_copy(rows_v.at[slot], num_sh.at[idx_d.at[slot]],
                         sem_s.at[par], add=True)
        pltpu.async_copy(ee_v.at[slot], den_sh.at[idx_d.at[slot]],
                         sem_s.at[par], add=True)

    def wait_scatter(par):
        pltpu.make_async_copy(z2_hbm.at[pl.ds(0, WIN)], rows_v.at[0],
                              sem_s.at[par]).wait()
        pltpu.make_async_copy(z1_hbm.at[pl.ds(0, WIN)], ee_v.at[0],
                              sem_s.at[par]).wait()

    def compute_ee(slot):
        def grp(g, c2):
            sv = idx_s[slot, pl.ds(g * L, L)]
            dv = idx_d[slot, pl.ds(g * L, L)]
            a = plsc.load_gather(ssrc_t, [sv])
            bb = plsc.load_gather(sdst_t, [dv])
            cc = _leaky(smax_v + bb)
            ee_v[slot, pl.ds(g * L, L)] = jnp.exp(_leaky(a + bb) - cc)
            return c2

        lax.fori_loop(0, WIN // L, grp, 0)

    def scale_rows(slot):
        def grp(g, c2):
            ev = ee_v[slot, pl.ds(g * L, L)]
            for j in range(L):
                w = jnp.full((L,), ev[j], jnp.float32)
                i = g * L + j
                for f in range(D // L):
                    rows_v[slot, i, pl.ds(f * L, L)] = (
                        rows_v[slot, i, pl.ds(f * L, L)] * w)
            return c2

        lax.fori_loop(0, WIN // L, grp, 0)

    fetch_idx(0, 0)
    fetch_idx(1, 1)
    wait_idx(0)
    start_gather(0, 0)
    wait_idx(1)
    start_gather(1, 1)

    def step(wp, carry):
        last_wp = wp >= NWIN // 4 - 1
        for b4 in range(4):
            par = b4 % 2
            fslot = (b4 + 2) % 4

            def drain():
                wait_scatter(par)

            if b4 <= 1:
                pl.when(wp >= 1)(drain)
            else:
                drain()

            def prefetch():
                fetch_idx(wp * 4 + b4 + 2, fslot)

            if b4 <= 1:
                prefetch()
            else:
                pl.when(jnp.logical_not(last_wp))(prefetch)

            compute_ee(b4)
            wait_gather(par)
            scale_rows(b4)
            start_scatter(b4, par)

            def advance():
                wait_idx(fslot)
                start_gather(fslot, par)

            if b4 <= 1:
                advance()
            else:
                pl.when(jnp.logical_not(last_wp))(advance)
        return carry

    lax.fori_loop(0, NWIN // 4, step, 0)
    wait_scatter(0)
    wait_scatter(1)
    plsc.subcore_barrier()

    for k in range(_ZCH):
        w = WIN if k < _ZCH - 1 else _ZREM
        pltpu.sync_copy(num_sh.at[pl.ds(sid * RSL + k * WIN, w)],
                        rows_v.at[0, pl.ds(0, w)])
        pltpu.sync_copy(rows_v.at[0, pl.ds(0, w)],
                        num_out.at[cid, pl.ds(sid * RSL + k * WIN, w)])
    pltpu.sync_copy(den_sh.at[pl.ds(sid * RSL, RSL)], zbuf)
    pltpu.sync_copy(zbuf, den_out.at[pl.ds(cid * NPAD + sid * RSL, RSL)])


_sc_layer = pl.kernel(
    _sc_body,
    out_type=[
        jax.ShapeDtypeStruct((NC, NPAD, D), jnp.float32),
        jax.ShapeDtypeStruct((NC * NPAD,), jnp.float32),
    ],
    mesh=plsc.VectorSubcoreMesh(core_axis_name="c", subcore_axis_name="s",
                                num_cores=NC, num_subcores=NS),
    compiler_params=pltpu.CompilerParams(needs_layout_passes=False),
    scratch_types=[
        pltpu.VMEM((NPAD,), jnp.float32),
        pltpu.VMEM((NPAD,), jnp.float32),
        pltpu.VMEM((L,), jnp.float32),
        pltpu.VMEM((4, WIN), jnp.int32),
        pltpu.VMEM((4, WIN), jnp.int32),
        pltpu.VMEM((4, WIN), jnp.float32),
        pltpu.VMEM((4, WIN, D), jnp.float32),
        pltpu.VMEM((RSL,), jnp.float32),
        pltpu.VMEM_SHARED((NPAD, D), jnp.float32),
        pltpu.VMEM_SHARED((NPAD,), jnp.float32),
        pltpu.SemaphoreType.DMA((4,)),
        pltpu.SemaphoreType.DMA((2,)),
        pltpu.SemaphoreType.DMA((2,)),
    ],
)


def _row_mask():
    rows = lax.broadcasted_iota(jnp.int32, (NPAD, 1), 0)
    return rows < N


def _attn_tables(h, asrc, adst):
    mask = _row_mask()
    ssrc = jnp.sum(h * asrc[None, :], axis=-1, keepdims=True)
    sdst = jnp.sum(h * adst[None, :], axis=-1, keepdims=True)
    ssrc = jnp.where(mask, ssrc, NEG)
    sdst = jnp.where(mask, sdst, 0.0)
    smax = jnp.max(ssrc)
    return ssrc, sdst, jnp.full((1, D), smax, jnp.float32)


def _tc_pre1_body(x_ref, w_ref, asrc_ref, adst_ref,
                  h_ref, ssrc_ref, sdst_ref, smax_ref):
    x = x_ref[...]
    h = jnp.dot(x, w_ref[...], preferred_element_type=jnp.float32)
    h_ref[...] = h
    ssrc, sdst, smax = _attn_tables(h, asrc_ref[...], adst_ref[...])
    ssrc_ref[...] = ssrc
    sdst_ref[...] = sdst
    smax_ref[...] = smax


def _gat_combine(num_ref, den_ref, b):
    num = num_ref[0] + num_ref[1]
    den = den_ref[0] + den_ref[1]
    gat = num / (den + 1e-16) + b[None, :]
    return jnp.maximum(gat, 0.0)


def _bn(x, g, b, m, v):
    return (x - m[None, :]) / jnp.sqrt(v[None, :] + 1e-5) * g[None, :] + b[None, :]


def _pre_next(xi, w_ref, asrc_ref, adst_ref, newprev_ref, h_ref, ssrc_ref,
              sdst_ref, smax_ref):
    xi = jnp.where(_row_mask(), xi, 0.0)
    newprev_ref[...] = xi
    h = jnp.dot(xi, w_ref[...], preferred_element_type=jnp.float32)
    h_ref[...] = h
    ssrc, sdst, smax = _attn_tables(h, asrc_ref[...], adst_ref[...])
    ssrc_ref[...] = ssrc
    sdst_ref[...] = sdst
    smax_ref[...] = smax


def _tc_mid1_body(num_ref, den_ref, cb_ref, bng_ref, bnb_ref, bnm_ref,
                  bnv_ref, w_ref, asrc_ref, adst_ref,
                  newprev_ref, h_ref, ssrc_ref, sdst_ref, smax_ref):
    act = _gat_combine(num_ref, den_ref, cb_ref[...])
    xi = _bn(act, bng_ref[...], bnb_ref[...], bnm_ref[...], bnv_ref[...])
    _pre_next(xi, w_ref, asrc_ref, adst_ref, newprev_ref, h_ref, ssrc_ref,
              sdst_ref, smax_ref)


def _tc_mid_body(num_ref, den_ref, prev_ref, cb_ref, bng_ref, bnb_ref,
                 bnm_ref, bnv_ref, pw_ref, pb_ref, w_ref, asrc_ref, adst_ref,
                 newprev_ref, h_ref, ssrc_ref, sdst_ref, smax_ref):
    act = _gat_combine(num_ref, den_ref, cb_ref[...])
    xi = _bn(act, bng_ref[...], bnb_ref[...], bnm_ref[...], bnv_ref[...])
    xi = xi + jnp.dot(prev_ref[...], pw_ref[...],
                      preferred_element_type=jnp.float32) + pb_ref[...][None, :]
    _pre_next(xi, w_ref, asrc_ref, adst_ref, newprev_ref, h_ref, ssrc_ref,
              sdst_ref, smax_ref)


def _tc_final_body(num_ref, den_ref, prev_ref, cb_ref, bng_ref, bnb_ref,
                   bnm_ref, bnv_ref, pw_ref, pb_ref, hw1_ref, hb1_ref,
                   hg_ref, hbb_ref, hm_ref, hv_ref, hw2_ref, hb2_ref,
                   out_ref):
    act = _gat_combine(num_ref, den_ref, cb_ref[...])
    xi = _bn(act, bng_ref[...], bnb_ref[...], bnm_ref[...], bnv_ref[...])
    xi = xi + jnp.dot(prev_ref[...], pw_ref[...],
                      preferred_element_type=jnp.float32) + pb_ref[...][None, :]
    xi = jnp.where(_row_mask(), xi, 0.0)
    g = jnp.sum(xi, axis=0, keepdims=True) / float(N)
    hh = jnp.dot(g, hw1_ref[...], preferred_element_type=jnp.float32)
    hh = jnp.maximum(hh + hb1_ref[...][None, :], 0.0)
    hh = _bn(hh, hg_ref[...], hbb_ref[...], hm_ref[...], hv_ref[...])
    out = jnp.dot(hh, hw2_ref[...], preferred_element_type=jnp.float32)
    out_ref[...] = out + hb2_ref[...][None, :]


_node_f32 = jax.ShapeDtypeStruct((NPAD, 1), jnp.float32)
_feat_f32 = jax.ShapeDtypeStruct((NPAD, D), jnp.float32)
_smax_f32 = jax.ShapeDtypeStruct((1, D), jnp.float32)

_tc_pre1 = pl.pallas_call(
    _tc_pre1_body,
    out_shape=[_feat_f32, _node_f32, _node_f32, _smax_f32],
)

_tc_mid1 = pl.pallas_call(
    _tc_mid1_body,
    out_shape=[_feat_f32, _feat_f32, _node_f32, _node_f32, _smax_f32],
)

_tc_mid = pl.pallas_call(
    _tc_mid_body,
    out_shape=[_feat_f32, _feat_f32, _node_f32, _node_f32, _smax_f32],
)

_tc_final = pl.pallas_call(
    _tc_final_body,
    out_shape=jax.ShapeDtypeStruct((1, 1), jnp.float32),
)


def kernel(x, edge_index, params):
    p = params
    xp = jnp.pad(x, ((0, NPAD - N), (0, 0)))
    src = edge_index[0]
    dst = edge_index[1]
    npad_e = EP - E
    pad_idx = N + (jnp.arange(npad_e, dtype=jnp.int32) % (NPAD - N))
    srcp = jnp.concatenate([src.astype(jnp.int32), pad_idx])
    dstp = jnp.concatenate([dst.astype(jnp.int32), pad_idx])
    z2 = jnp.zeros((NPAD, D), jnp.float32)
    z1 = jnp.zeros((NPAD,), jnp.float32)

    def flat(a):
        return a.reshape(NPAD)

    def d2(a):
        return a.reshape(NC, NPAD, 1)

    h, ssrc, sdst, smax = _tc_pre1(xp, p['conv1_W'], p['conv1_asrc'],
                                   p['conv1_adst'])
    num, den = _sc_layer(h, flat(ssrc), flat(sdst), smax.reshape(D), srcp,
                         dstp, z2, z1)

    prev, h, ssrc, sdst, smax = _tc_mid1(
        num, d2(den), p['conv1_b'],
        p['bn1_g'], p['bn1_b'], p['bn1_m'], p['bn1_v'],
        p['conv2_W'], p['conv2_asrc'], p['conv2_adst'])
    num, den = _sc_layer(h, flat(ssrc), flat(sdst), smax.reshape(D), srcp,
                         dstp, z2, z1)

    for i in range(3, 6):
        j = i - 1
        prev, h, ssrc, sdst, smax = _tc_mid(
            num, d2(den), prev, p['conv%d_b' % j],
            p['bn%d_g' % j], p['bn%d_b' % j], p['bn%d_m' % j], p['bn%d_v' % j],
            p['proj%d_W' % j], p['proj%d_b' % j],
            p['conv%d_W' % i], p['conv%d_asrc' % i], p['conv%d_adst' % i])
        num, den = _sc_layer(h, flat(ssrc), flat(sdst), smax.reshape(D), srcp,
                             dstp, z2, z1)

    out = _tc_final(
        num, d2(den), prev, p['conv5_b'],
        p['bn5_g'], p['bn5_b'], p['bn5_m'], p['bn5_v'],
        p['proj5_W'], p['proj5_b'],
        p['head_W1'], p['head_b1'],
        p['headbn_g'], p['headbn_b'], p['headbn_m'], p['headbn_v'],
        p['head_W2'], p['head_b2'])
    return out.reshape(-1)


# speedup vs baseline: 1.3052x; 1.0415x over previous
import jax
import jax.numpy as jnp
from jax import lax
from jax.experimental import pallas as pl
from jax.experimental.pallas import tpu as pltpu
from jax.experimental.pallas import tpu_sc as plsc

N = 10000
D = 128
E = 320000
NC = 2
NS = 16
L = 16
WIN = 96
NPAD = N + 112
NW = NC * NS
NWIN = 108
EPW = NWIN * WIN
EP = EPW * NW
RSL = NPAD // NS
NEG = -1.0e9


def _leaky(z):
    return jnp.maximum(z, 0.2 * z)


_ZCH = (RSL + WIN - 1) // WIN
_ZREM = RSL - (_ZCH - 1) * WIN


def _sc_body(h_hbm, ssrc_hbm, sdst_hbm, smax_hbm, srcp_hbm, dstp_hbm,
             z2_hbm, z1_hbm, num_out, den_out,
             ssrc_t, sdst_t, smax_t, idx_s, idx_d, ee_v, rows_v, zbuf,
             num_sh, den_sh, sem_i, sem_g, sem_s, sem_e):
    cid = lax.axis_index("c")
    sid = lax.axis_index("s")
    wid = sid * NC + cid

    pltpu.sync_copy(ssrc_hbm, ssrc_t)
    pltpu.sync_copy(sdst_hbm, sdst_t)
    pltpu.sync_copy(smax_hbm.at[pl.ds(0, L)], smax_t)

    pltpu.sync_copy(z1_hbm.at[pl.ds(sid * RSL, RSL)], zbuf)
    pltpu.sync_copy(zbuf, den_sh.at[pl.ds(sid * RSL, RSL)])
    pltpu.sync_copy(z2_hbm.at[pl.ds(0, WIN)], rows_v.at[0])
    for k in range(_ZCH):
        w = WIN if k < _ZCH - 1 else _ZREM
        pltpu.sync_copy(rows_v.at[0, pl.ds(0, w)],
                        num_sh.at[pl.ds(sid * RSL + k * WIN, w)])
    plsc.subcore_barrier()

    base = wid * EPW
    smax_v = smax_t[...]

    def fetch_idx(wi, slot):
        off = base + wi * WIN
        pltpu.async_copy(srcp_hbm.at[pl.ds(off, WIN)], idx_s.at[slot],
                         sem_i.at[slot])
        pltpu.async_copy(dstp_hbm.at[pl.ds(off, WIN)], idx_d.at[slot],
                         sem_i.at[slot])

    def wait_idx(slot):
        pltpu.make_async_copy(srcp_hbm.at[pl.ds(0, WIN)], idx_s.at[slot],
                              sem_i.at[slot]).wait()
        pltpu.make_async_copy(dstp_hbm.at[pl.ds(0, WIN)], idx_d.at[slot],
                              sem_i.at[slot]).wait()

    def start_gather(slot, b):
        pltpu.async_copy(h_hbm.at[idx_s.at[slot]], rows_v.at[b], sem_g)

    def wait_gather():
        pltpu.make_async_copy(z2_hbm.at[pl.ds(0, WIN)], rows_v.at[0],
                              sem_g).wait()

    def start_scatter_rows(slot, b):
        pltpu.async_copy(rows_v.at[b], num_sh.at[idx_d.at[slot]], sem_s,
                         add=True)

    def start_scatter_ee(slot, b):
        pltpu.async_copy(ee_v.at[b], den_sh.at[idx_d.at[slot]],
                         sem_e.at[b], add=True)

    def wait_scatter(epar):
        pltpu.make_async_copy(z2_hbm.at[pl.ds(0, WIN)], rows_v.at[0],
                              sem_s).wait()
        pltpu.make_async_copy(z1_hbm.at[pl.ds(0, WIN)], ee_v.at[0],
                              sem_e.at[epar]).wait()

    def compute_ee(slot, b):
        def grp(g, c2):
            sv = idx_s[slot, pl.ds(g * L, L)]
            dv = idx_d[slot, pl.ds(g * L, L)]
            a = plsc.load_gather(ssrc_t, [sv])
            bb = plsc.load_gather(sdst_t, [dv])
            cc = _leaky(smax_v + bb)
            ee_v[b, pl.ds(g * L, L)] = jnp.exp(_leaky(a + bb) - cc)
            return c2

        lax.fori_loop(0, WIN // L, grp, 0)

    def scale_rows(b):
        def grp(g, c2):
            ev = ee_v[b, pl.ds(g * L, L)]
            for j in range(L):
                w = jnp.full((L,), ev[j], jnp.float32)
                i = g * L + j
                for f in range(D // L):
                    rows_v[b, i, pl.ds(f * L, L)] = (
                        rows_v[b, i, pl.ds(f * L, L)] * w)
            return c2

        lax.fori_loop(0, WIN // L, grp, 0)

    fetch_idx(0, 0)
    fetch_idx(1, 1)
    wait_idx(0)
    start_gather(0, 0)

    def step(wp, carry):
        last_wp = wp >= NWIN // 4 - 1
        for b4 in range(4):
            b = b4 % 2
            nb = 1 - b
            nslot = (b4 + 1) % 4
            fslot = (b4 + 2) % 4
            compute_ee(b4, b)
            start_scatter_ee(b4, b)

            def drain():
                wait_scatter(nb)

            if b4 == 0:
                pl.when(wp >= 1)(drain)
            else:
                drain()

            def advance():
                wait_idx(nslot)
                start_gather(nslot, nb)

            if b4 == 3:
                pl.when(jnp.logical_not(last_wp))(advance)
            else:
                advance()

            def prefetch():
                fetch_idx(wp * 4 + b4 + 2, fslot)

            if b4 >= 2:
                pl.when(jnp.logical_not(last_wp))(prefetch)
            else:
                prefetch()

            wait_gather()
            scale_rows(b)
            start_scatter_rows(b4, b)
        return carry

    lax.fori_loop(0, NWIN // 4, step, 0)
    wait_scatter((NWIN - 1) % 2)
    plsc.subcore_barrier()

    def _w(k):
        return WIN if k < _ZCH - 1 else _ZREM

    for k in range(_ZCH):
        pltpu.sync_copy(num_sh.at[pl.ds(sid * RSL + k * WIN, _w(k))],
                        rows_v.at[k % 2, pl.ds(0, _w(k))])
        if k >= 1:
            pltpu.make_async_copy(z2_hbm.at[pl.ds(0, _w(k - 1))],
                                  rows_v.at[0, pl.ds(0, _w(k - 1))],
                                  sem_s).wait()
        pltpu.async_copy(rows_v.at[k % 2, pl.ds(0, _w(k))],
                         num_out.at[cid, pl.ds(sid * RSL + k * WIN, _w(k))],
                         sem_s)
    pltpu.sync_copy(den_sh.at[pl.ds(sid * RSL, RSL)], zbuf)
    pltpu.sync_copy(zbuf, den_out.at[pl.ds(cid * NPAD + sid * RSL, RSL)])
    pltpu.make_async_copy(z2_hbm.at[pl.ds(0, _w(_ZCH - 1))],
                          rows_v.at[0, pl.ds(0, _w(_ZCH - 1))],
                          sem_s).wait()


_sc_layer = pl.kernel(
    _sc_body,
    out_type=[
        jax.ShapeDtypeStruct((NC, NPAD, D), jnp.float32),
        jax.ShapeDtypeStruct((NC * NPAD,), jnp.float32),
    ],
    mesh=plsc.VectorSubcoreMesh(core_axis_name="c", subcore_axis_name="s",
                                num_cores=NC, num_subcores=NS),
    compiler_params=pltpu.CompilerParams(needs_layout_passes=False),
    scratch_types=[
        pltpu.VMEM((NPAD,), jnp.float32),
        pltpu.VMEM((NPAD,), jnp.float32),
        pltpu.VMEM((L,), jnp.float32),
        pltpu.VMEM((4, WIN), jnp.int32),
        pltpu.VMEM((4, WIN), jnp.int32),
        pltpu.VMEM((2, WIN), jnp.float32),
        pltpu.VMEM((2, WIN, D), jnp.float32),
        pltpu.VMEM((RSL,), jnp.float32),
        pltpu.VMEM_SHARED((NPAD, D), jnp.float32),
        pltpu.VMEM_SHARED((NPAD,), jnp.float32),
        pltpu.SemaphoreType.DMA((4,)),
        pltpu.SemaphoreType.DMA,
        pltpu.SemaphoreType.DMA,
        pltpu.SemaphoreType.DMA((2,)),
    ],
)


def _row_mask():
    rows = lax.broadcasted_iota(jnp.int32, (NPAD, 1), 0)
    return rows < N


def _attn_tables(h, asrc, adst):
    mask = _row_mask()
    ssrc = jnp.sum(h * asrc[None, :], axis=-1, keepdims=True)
    sdst = jnp.sum(h * adst[None, :], axis=-1, keepdims=True)
    ssrc = jnp.where(mask, ssrc, NEG)
    sdst = jnp.where(mask, sdst, 0.0)
    smax = jnp.max(ssrc)
    return ssrc, sdst, jnp.full((1, D), smax, jnp.float32)


def _tc_pre1_body(x_ref, w_ref, asrc_ref, adst_ref,
                  h_ref, ssrc_ref, sdst_ref, smax_ref):
    x = x_ref[...]
    h = jnp.dot(x, w_ref[...], preferred_element_type=jnp.float32)
    h_ref[...] = h
    ssrc, sdst, smax = _attn_tables(h, asrc_ref[...], adst_ref[...])
    ssrc_ref[...] = ssrc
    sdst_ref[...] = sdst
    smax_ref[...] = smax


def _gat_combine(num_ref, den_ref, b):
    num = num_ref[0] + num_ref[1]
    den = den_ref[0] + den_ref[1]
    gat = num / (den + 1e-16) + b[None, :]
    return jnp.maximum(gat, 0.0)


def _bn(x, g, b, m, v):
    return (x - m[None, :]) / jnp.sqrt(v[None, :] + 1e-5) * g[None, :] + b[None, :]


def _pre_next(xi, w_ref, asrc_ref, adst_ref, newprev_ref, h_ref, ssrc_ref,
              sdst_ref, smax_ref):
    xi = jnp.where(_row_mask(), xi, 0.0)
    newprev_ref[...] = xi
    h = jnp.dot(xi, w_ref[...], preferred_element_type=jnp.float32)
    h_ref[...] = h
    ssrc, sdst, smax = _attn_tables(h, asrc_ref[...], adst_ref[...])
    ssrc_ref[...] = ssrc
    sdst_ref[...] = sdst
    smax_ref[...] = smax


def _tc_mid1_body(num_ref, den_ref, cb_ref, bng_ref, bnb_ref, bnm_ref,
                  bnv_ref, w_ref, asrc_ref, adst_ref,
                  newprev_ref, h_ref, ssrc_ref, sdst_ref, smax_ref):
    act = _gat_combine(num_ref, den_ref, cb_ref[...])
    xi = _bn(act, bng_ref[...], bnb_ref[...], bnm_ref[...], bnv_ref[...])
    _pre_next(xi, w_ref, asrc_ref, adst_ref, newprev_ref, h_ref, ssrc_ref,
              sdst_ref, smax_ref)


def _tc_mid_body(num_ref, den_ref, prev_ref, cb_ref, bng_ref, bnb_ref,
                 bnm_ref, bnv_ref, pw_ref, pb_ref, w_ref, asrc_ref, adst_ref,
                 newprev_ref, h_ref, ssrc_ref, sdst_ref, smax_ref):
    act = _gat_combine(num_ref, den_ref, cb_ref[...])
    xi = _bn(act, bng_ref[...], bnb_ref[...], bnm_ref[...], bnv_ref[...])
    xi = xi + jnp.dot(prev_ref[...], pw_ref[...],
                      preferred_element_type=jnp.float32) + pb_ref[...][None, :]
    _pre_next(xi, w_ref, asrc_ref, adst_ref, newprev_ref, h_ref, ssrc_ref,
              sdst_ref, smax_ref)


def _tc_final_body(num_ref, den_ref, prev_ref, cb_ref, bng_ref, bnb_ref,
                   bnm_ref, bnv_ref, pw_ref, pb_ref, hw1_ref, hb1_ref,
                   hg_ref, hbb_ref, hm_ref, hv_ref, hw2_ref, hb2_ref,
                   out_ref):
    act = _gat_combine(num_ref, den_ref, cb_ref[...])
    xi = _bn(act, bng_ref[...], bnb_ref[...], bnm_ref[...], bnv_ref[...])
    xi = xi + jnp.dot(prev_ref[...], pw_ref[...],
                      preferred_element_type=jnp.float32) + pb_ref[...][None, :]
    xi = jnp.where(_row_mask(), xi, 0.0)
    g = jnp.sum(xi, axis=0, keepdims=True) / float(N)
    hh = jnp.dot(g, hw1_ref[...], preferred_element_type=jnp.float32)
    hh = jnp.maximum(hh + hb1_ref[...][None, :], 0.0)
    hh = _bn(hh, hg_ref[...], hbb_ref[...], hm_ref[...], hv_ref[...])
    out = jnp.dot(hh, hw2_ref[...], preferred_element_type=jnp.float32)
    out_ref[...] = out + hb2_ref[...][None, :]


_node_f32 = jax.ShapeDtypeStruct((NPAD, 1), jnp.float32)
_feat_f32 = jax.ShapeDtypeStruct((NPAD, D), jnp.float32)
_smax_f32 = jax.ShapeDtypeStruct((1, D), jnp.float32)

_tc_pre1 = pl.pallas_call(
    _tc_pre1_body,
    out_shape=[_feat_f32, _node_f32, _node_f32, _smax_f32],
)

_tc_mid1 = pl.pallas_call(
    _tc_mid1_body,
    out_shape=[_feat_f32, _feat_f32, _node_f32, _node_f32, _smax_f32],
)

_tc_mid = pl.pallas_call(
    _tc_mid_body,
    out_shape=[_feat_f32, _feat_f32, _node_f32, _node_f32, _smax_f32],
)

_tc_final = pl.pallas_call(
    _tc_final_body,
    out_shape=jax.ShapeDtypeStruct((1, 1), jnp.float32),
)


def kernel(x, edge_index, params):
    p = params
    xp = jnp.pad(x, ((0, NPAD - N), (0, 0)))
    src = edge_index[0]
    dst = edge_index[1]
    npad_e = EP - E
    pad_idx = N + (jnp.arange(npad_e, dtype=jnp.int32) % (NPAD - N))
    srcp = jnp.concatenate([src.astype(jnp.int32), pad_idx])
    dstp = jnp.concatenate([dst.astype(jnp.int32), pad_idx])
    z2 = jnp.zeros((NPAD, D), jnp.float32)
    z1 = jnp.zeros((NPAD,), jnp.float32)

    def flat(a):
        return a.reshape(NPAD)

    def d2(a):
        return a.reshape(NC, NPAD, 1)

    h, ssrc, sdst, smax = _tc_pre1(xp, p['conv1_W'], p['conv1_asrc'],
                                   p['conv1_adst'])
    num, den = _sc_layer(h, flat(ssrc), flat(sdst), smax.reshape(D), srcp,
                         dstp, z2, z1)

    prev, h, ssrc, sdst, smax = _tc_mid1(
        num, d2(den), p['conv1_b'],
        p['bn1_g'], p['bn1_b'], p['bn1_m'], p['bn1_v'],
        p['conv2_W'], p['conv2_asrc'], p['conv2_adst'])
    num, den = _sc_layer(h, flat(ssrc), flat(sdst), smax.reshape(D), srcp,
                         dstp, z2, z1)

    for i in range(3, 6):
        j = i - 1
        prev, h, ssrc, sdst, smax = _tc_mid(
            num, d2(den), prev, p['conv%d_b' % j],
            p['bn%d_g' % j], p['bn%d_b' % j], p['bn%d_m' % j], p['bn%d_v' % j],
            p['proj%d_W' % j], p['proj%d_b' % j],
            p['conv%d_W' % i], p['conv%d_asrc' % i], p['conv%d_adst' % i])
        num, den = _sc_layer(h, flat(ssrc), flat(sdst), smax.reshape(D), srcp,
                             dstp, z2, z1)

    out = _tc_final(
        num, d2(den), prev, p['conv5_b'],
        p['bn5_g'], p['bn5_b'], p['bn5_m'], p['bn5_v'],
        p['proj5_W'], p['proj5_b'],
        p['head_W1'], p['head_b1'],
        p['headbn_g'], p['headbn_b'], p['headbn_m'], p['headbn_v'],
        p['head_W2'], p['head_b2'])
    return out.reshape(-1)
